# Initial kernel scaffold; baseline (speedup 1.0000x reference)
import jax
import jax.numpy as jnp
from jax.experimental import pallas as pl


def kernel(vis_fs, lan_fs, tag_fs):
    raise NotImplementedError("write your pallas kernel here")


# speedup vs baseline: 35.5258x; 35.5258x over previous
import functools

import jax
import jax.numpy as jnp
from jax.experimental import pallas as pl
from jax.experimental.pallas import tpu as pltpu

_B = 64
_V = 2048
_D = 64
_A_BLK = 8
_N_STEPS = _B // _A_BLK

_NEG = float("-inf")


def _top2_lastaxis(s):
    m1 = jnp.max(s, axis=1, keepdims=True)
    eq = s == m1
    cnt = jnp.sum(eq.astype(jnp.int32), axis=1, keepdims=True)
    m2 = jnp.max(jnp.where(eq, _NEG, s), axis=1, keepdims=True)
    m2 = jnp.where(cnt > 1, m1, m2)
    return m1, m2


def _rank_select_mask(s, rank):
    x = s[:, :, None]
    y = s[:, None, :]
    cnt_gt = jnp.sum((y > x).astype(jnp.int32), axis=2)
    cnt_ge = cnt_gt + jnp.sum((y == x).astype(jnp.int32), axis=2)
    cond = jnp.logical_and(cnt_gt <= rank, rank < cnt_ge)
    val = jnp.max(jnp.where(cond, s, _NEG), axis=1, keepdims=True)
    a_iota = jax.lax.broadcasted_iota(jnp.int32, s.shape, 1)
    first = jnp.min(jnp.where(cond, a_iota, _B), axis=1, keepdims=True)
    mask = a_iota == first
    return mask, val


def _loss_kernel(vis_ref, lan_ref, tag0_ref, tag1_ref, out_ref,
                 vl0_s, vl1_s):
    i = pl.program_id(0)
    lan = lan_ref[...]

    m1s, m2s = [], []
    for j in range(_A_BLK):
        s = jax.lax.dot_general(
            lan, vis_ref[j],
            (((1,), (1,)), ((), ())),
            preferred_element_type=jnp.float32)
        m1, m2 = _top2_lastaxis(s)
        m1s.append(m1)
        m2s.append(m2)
    vl0_s[pl.ds(i, 1)] = jnp.concatenate(m1s, axis=1)[None]
    vl1_s[pl.ds(i, 1)] = jnp.concatenate(m2s, axis=1)[None]

    @pl.when(i == _N_STEPS - 1)
    def _finalize():
        vl0 = vl0_s[...]
        vl1 = vl1_s[...]
        s_iota = jax.lax.broadcasted_iota(jnp.int32, vl0.shape, 0)
        b_iota = jax.lax.broadcasted_iota(jnp.int32, vl0.shape, 1)
        j_iota = jax.lax.broadcasted_iota(jnp.int32, vl0.shape, 2)
        diag = (s_iota * _A_BLK + j_iota) == b_iota

        picked_v = jnp.sum(jnp.where(diag, vl0, 0.0), axis=(0, 2))
        vl1_m = jnp.where(diag, _NEG, vl1)
        m_v = jnp.maximum(jnp.max(vl0, axis=(0, 2)), jnp.max(vl1_m, axis=(0, 2)))
        se_v = (jnp.sum(jnp.exp(vl0 - m_v[None, :, None]), axis=(0, 2))
                + jnp.sum(jnp.exp(vl1_m - m_v[None, :, None]), axis=(0, 2)))
        loss_v = jnp.mean(m_v + jnp.log(se_v) - picked_v)

        s0 = jax.lax.dot_general(lan, tag0_ref[...], (((1,), (1,)), ((), ())),
                                 preferred_element_type=jnp.float32)
        s1 = jax.lax.dot_general(lan, tag1_ref[...], (((1,), (1,)), ((), ())),
                                 preferred_element_type=jnp.float32)
        rank = jax.lax.broadcasted_iota(jnp.int32, (_B, _B), 0)
        _, picked_t = _rank_select_mask(s0, rank)
        rm_mask, _ = _rank_select_mask(s1, rank)
        s1_m = jnp.where(rm_mask, _NEG, s1)
        m_t = jnp.maximum(jnp.max(s0, axis=1, keepdims=True),
                          jnp.max(s1_m, axis=1, keepdims=True))
        se_t = (jnp.sum(jnp.exp(s0 - m_t), axis=1, keepdims=True)
                + jnp.sum(jnp.exp(s1_m - m_t), axis=1, keepdims=True))
        loss_t = jnp.mean(m_t + jnp.log(se_t) - picked_t)

        out_ref[...] = (loss_v + loss_t)[None, None]


@functools.partial(jax.jit, static_argnames=("interpret",))
def _run(vis_fs, lan_fs, tag_fs, interpret=False):
    lan = lan_fs.reshape(_B, _D)
    tag0 = tag_fs[:, 0, :]
    tag1 = tag_fs[:, 1, :]
    out = pl.pallas_call(
        _loss_kernel,
        grid=(_N_STEPS,),
        in_specs=[
            pl.BlockSpec((_A_BLK, _V, _D), lambda i: (i, 0, 0)),
            pl.BlockSpec((_B, _D), lambda i: (0, 0)),
            pl.BlockSpec((_B, _D), lambda i: (0, 0)),
            pl.BlockSpec((_B, _D), lambda i: (0, 0)),
        ],
        out_specs=pl.BlockSpec((1, 1), lambda i: (0, 0)),
        out_shape=jax.ShapeDtypeStruct((1, 1), jnp.float32),
        scratch_shapes=[
            pltpu.VMEM((_N_STEPS, _B, _A_BLK), jnp.float32),
            pltpu.VMEM((_N_STEPS, _B, _A_BLK), jnp.float32),
        ],
        interpret=interpret,
    )(vis_fs, lan, tag0, tag1)
    return out[0, 0]


def kernel(vis_fs, lan_fs, tag_fs):
    return _run(vis_fs, lan_fs, tag_fs)


# speedup vs baseline: 39.8594x; 1.1220x over previous
import functools

import jax
import jax.numpy as jnp
from jax.experimental import pallas as pl
from jax.experimental.pallas import tpu as pltpu

_B = 64
_V = 2048
_D = 64
_A_BLK = 8
_N_STEPS = _B // _A_BLK

_NEG = float("-inf")


_LANES = 128
_N_CHUNK = _V // _LANES


def _top2_lastaxis(s):
    c0 = s[:, 0:_LANES]
    c1 = s[:, _LANES:2 * _LANES]
    hi = jnp.maximum(c0, c1)
    lo = jnp.minimum(c0, c1)
    for c in range(2, _N_CHUNK):
        x = s[:, c * _LANES:(c + 1) * _LANES]
        lo = jnp.maximum(lo, jnp.minimum(hi, x))
        hi = jnp.maximum(hi, x)
    m1 = jnp.max(hi, axis=1, keepdims=True)
    eq = hi == m1
    cnt = jnp.sum(jnp.where(eq, 1.0, 0.0), axis=1, keepdims=True)
    m2 = jnp.max(jnp.where(eq, lo, hi), axis=1, keepdims=True)
    m2 = jnp.where(cnt > 1.0, m1, m2)
    return m1, m2


def _rank_select(s0, s1):
    c = jnp.concatenate([s0, s1], axis=1)
    y = jnp.concatenate(
        [jnp.broadcast_to(s0[:, :, None], (_B, _B, _B)),
         jnp.broadcast_to(s1[:, :, None], (_B, _B, _B))], axis=2)
    x = c[:, None, :]
    gt = jnp.sum(jnp.where(y > x, 1.0, 0.0), axis=1)
    ge = jnp.sum(jnp.where(y >= x, 1.0, 0.0), axis=1)
    rank = jax.lax.broadcasted_iota(jnp.int32, (_B, 2 * _B), 0).astype(jnp.float32)
    cond = jnp.logical_and(gt <= rank, rank < ge)
    picked = jnp.max(jnp.where(cond[:, :_B], s0, _NEG), axis=1, keepdims=True)
    a_iota = jax.lax.broadcasted_iota(jnp.int32, (_B, _B), 1).astype(jnp.float32)
    first = jnp.min(jnp.where(cond[:, _B:], a_iota, float(_B)),
                    axis=1, keepdims=True)
    s1_m = jnp.where(a_iota == first, _NEG, s1)
    return picked, s1_m


def _loss_kernel(vis_ref, lan_ref, tag0_ref, tag1_ref, out_ref,
                 vl0_s, vl1_s):
    i = pl.program_id(0)
    lan = lan_ref[...]

    m1s, m2s = [], []
    for j in range(_A_BLK):
        s = jax.lax.dot_general(
            lan, vis_ref[j],
            (((1,), (1,)), ((), ())),
            preferred_element_type=jnp.float32)
        m1, m2 = _top2_lastaxis(s)
        m1s.append(m1)
        m2s.append(m2)
    vl0_s[pl.ds(i, 1)] = jnp.concatenate(m1s, axis=1)[None]
    vl1_s[pl.ds(i, 1)] = jnp.concatenate(m2s, axis=1)[None]

    @pl.when(i == _N_STEPS - 1)
    def _finalize():
        vl0 = vl0_s[...]
        vl1 = vl1_s[...]
        s_iota = jax.lax.broadcasted_iota(jnp.int32, vl0.shape, 0)
        b_iota = jax.lax.broadcasted_iota(jnp.int32, vl0.shape, 1)
        j_iota = jax.lax.broadcasted_iota(jnp.int32, vl0.shape, 2)
        diag = (s_iota * _A_BLK + j_iota) == b_iota

        picked_v = jnp.sum(jnp.where(diag, vl0, 0.0), axis=(0, 2))
        vl1_m = jnp.where(diag, _NEG, vl1)
        m_v = jnp.maximum(jnp.max(vl0, axis=(0, 2)), jnp.max(vl1_m, axis=(0, 2)))
        se_v = (jnp.sum(jnp.exp(vl0 - m_v[None, :, None]), axis=(0, 2))
                + jnp.sum(jnp.exp(vl1_m - m_v[None, :, None]), axis=(0, 2)))
        loss_v = jnp.mean(m_v + jnp.log(se_v) - picked_v)

        s0 = jax.lax.dot_general(lan, tag0_ref[...], (((1,), (1,)), ((), ())),
                                 preferred_element_type=jnp.float32)
        s1 = jax.lax.dot_general(lan, tag1_ref[...], (((1,), (1,)), ((), ())),
                                 preferred_element_type=jnp.float32)
        picked_t, s1_m = _rank_select(s0, s1)
        m_t = jnp.maximum(jnp.max(s0, axis=1, keepdims=True),
                          jnp.max(s1_m, axis=1, keepdims=True))
        se_t = (jnp.sum(jnp.exp(s0 - m_t), axis=1, keepdims=True)
                + jnp.sum(jnp.exp(s1_m - m_t), axis=1, keepdims=True))
        loss_t = jnp.mean(m_t + jnp.log(se_t) - picked_t)

        out_ref[...] = (loss_v + loss_t)[None, None]


@functools.partial(jax.jit, static_argnames=("interpret",))
def _run(vis_fs, lan_fs, tag_fs, interpret=False):
    lan = lan_fs.reshape(_B, _D)
    tag0 = tag_fs[:, 0, :]
    tag1 = tag_fs[:, 1, :]
    out = pl.pallas_call(
        _loss_kernel,
        grid=(_N_STEPS,),
        in_specs=[
            pl.BlockSpec((_A_BLK, _V, _D), lambda i: (i, 0, 0)),
            pl.BlockSpec((_B, _D), lambda i: (0, 0)),
            pl.BlockSpec((_B, _D), lambda i: (0, 0)),
            pl.BlockSpec((_B, _D), lambda i: (0, 0)),
        ],
        out_specs=pl.BlockSpec((1, 1), lambda i: (0, 0)),
        out_shape=jax.ShapeDtypeStruct((1, 1), jnp.float32),
        scratch_shapes=[
            pltpu.VMEM((_N_STEPS, _B, _A_BLK), jnp.float32),
            pltpu.VMEM((_N_STEPS, _B, _A_BLK), jnp.float32),
        ],
        interpret=interpret,
    )(vis_fs, lan, tag0, tag1)
    return out[0, 0]


def kernel(vis_fs, lan_fs, tag_fs):
    return _run(vis_fs, lan_fs, tag_fs)
